# baseline (speedup 1.0000x reference)
import jax
import jax.numpy as jnp
from jax.experimental import pallas as pl
from jax.experimental.pallas import tpu as pltpu

_BB = 8


def _body(idx_ref, x_ref, w_ref, o_ref):
    hw = x_ref.shape[-1]
    n_idx = idx_ref.shape[0]
    pos = jax.lax.broadcasted_iota(jnp.int32, (1, hw), 1)
    hits = (idx_ref[...] == pos).astype(jnp.float32)
    wt = jnp.sum(hits, axis=0, keepdims=True) * (1.0 / n_idx)
    pooled = jnp.sum(x_ref[...] * wt[None, :, :], axis=2)
    o_ref[...] = jax.lax.dot_general(
        pooled, w_ref[...], (((1,), (1,)), ((), ())),
        preferred_element_type=jnp.float32)


def kernel(x, W, indice):
    b, c, h, w = x.shape
    hw = h * w
    n, _ = W.shape
    x3 = x.reshape(b, c, hw)
    idx = indice.astype(jnp.int32).reshape(-1, 1)
    n_idx = idx.shape[0]

    grid = (b // _BB,)
    out = pl.pallas_call(
        _body,
        grid=grid,
        in_specs=[
            pl.BlockSpec((n_idx, 1), lambda i: (0, 0)),
            pl.BlockSpec((_BB, c, hw), lambda i: (i, 0, 0)),
            pl.BlockSpec((n, c), lambda i: (0, 0)),
        ],
        out_specs=pl.BlockSpec((_BB, n), lambda i: (i, 0)),
        out_shape=jax.ShapeDtypeStruct((b, n), jnp.float32),
        compiler_params=pltpu.CompilerParams(
            dimension_semantics=("arbitrary",)),
    )(idx, x3, W)
    return out


# speedup vs baseline: 1.0011x; 1.0011x over previous
import jax
import jax.numpy as jnp
from jax.experimental import pallas as pl
from jax.experimental.pallas import tpu as pltpu

_BB = 8
_NSTREAM = 4


def _body(idx_ref, *refs):
    x_refs = refs[:_NSTREAM]
    w_ref, o_ref = refs[_NSTREAM], refs[_NSTREAM + 1]
    hw = x_refs[0].shape[-1]
    n_idx = idx_ref.shape[0]
    pos = jax.lax.broadcasted_iota(jnp.int32, (1, hw), 1)
    hits = (idx_ref[...] == pos).astype(jnp.float32)
    wt = jnp.sum(hits, axis=0, keepdims=True) * (1.0 / n_idx)
    pooled = jnp.concatenate(
        [jnp.sum(xr[...] * wt[None, :, :], axis=2) for xr in x_refs],
        axis=1)
    o_ref[...] = jax.lax.dot_general(
        pooled, w_ref[...], (((1,), (1,)), ((), ())),
        preferred_element_type=jnp.float32)


def kernel(x, W, indice):
    b, c, h, w = x.shape
    hw = h * w
    n, _ = W.shape
    x3 = x.reshape(b, c, hw)
    idx = indice.astype(jnp.int32).reshape(-1, 1)
    n_idx = idx.shape[0]
    cs = c // _NSTREAM

    def _mk_spec(k):
        return pl.BlockSpec((_BB, cs, hw), lambda i, k=k: (i, k, 0))

    grid = (b // _BB,)
    out = pl.pallas_call(
        _body,
        grid=grid,
        in_specs=[
            pl.BlockSpec((n_idx, 1), lambda i: (0, 0)),
            *[_mk_spec(k) for k in range(_NSTREAM)],
            pl.BlockSpec((n, c), lambda i: (0, 0)),
        ],
        out_specs=pl.BlockSpec((_BB, n), lambda i: (i, 0)),
        out_shape=jax.ShapeDtypeStruct((b, n), jnp.float32),
        compiler_params=pltpu.CompilerParams(
            dimension_semantics=("arbitrary",)),
    )(idx, *([x3] * _NSTREAM), W)
    return out


# speedup vs baseline: 1.0022x; 1.0011x over previous
import jax
import jax.numpy as jnp
from jax.experimental import pallas as pl
from jax.experimental.pallas import tpu as pltpu

_BB = 8
_NSTREAM = 4


def _body(idx_ref, *refs):
    x_refs = refs[:_NSTREAM]
    w_ref, o_ref = refs[_NSTREAM], refs[_NSTREAM + 1]
    hw = x_refs[0].shape[-1]
    n_idx = idx_ref.shape[0]
    pos = jax.lax.broadcasted_iota(jnp.int32, (1, hw), 1)
    hits = (idx_ref[...] == pos).astype(jnp.float32)
    wt = jnp.sum(hits, axis=0, keepdims=True) * (1.0 / n_idx)
    pooled = jnp.concatenate(
        [jnp.sum(xr[...] * wt[None, :, :], axis=2) for xr in x_refs],
        axis=1)
    o_ref[...] = jax.lax.dot_general(
        pooled, w_ref[...], (((1,), (1,)), ((), ())),
        preferred_element_type=jnp.float32)


def kernel(x, W, indice):
    b, c, h, w = x.shape
    hw = h * w
    n, _ = W.shape
    x3 = x.reshape(b, c, hw)
    idx = indice.astype(jnp.int32).reshape(-1, 1)
    n_idx = idx.shape[0]
    cs = c // _NSTREAM

    def _mk_spec(k):
        return pl.BlockSpec((_BB, cs, hw), lambda i, k=k: (i, k, 0))

    grid = (b // _BB,)
    out = pl.pallas_call(
        _body,
        grid=grid,
        in_specs=[
            pl.BlockSpec((n_idx, 1), lambda i: (0, 0)),
            *[_mk_spec(k) for k in range(_NSTREAM)],
            pl.BlockSpec((n, c), lambda i: (0, 0)),
        ],
        out_specs=pl.BlockSpec((_BB, n), lambda i: (i, 0)),
        out_shape=jax.ShapeDtypeStruct((b, n), jnp.float32),
        compiler_params=pltpu.CompilerParams(
            dimension_semantics=("parallel",)),
    )(idx, *([x3] * _NSTREAM), W)
    return out
